# baseline (speedup 1.0000x reference)
import functools

import jax
import jax.numpy as jnp
from jax import lax
from jax.experimental import pallas as pl
from jax.experimental.pallas import tpu as pltpu
from jax.experimental.pallas import tpu_sc as plsc

B = 8
N = 16384
P = 27
S = 512
R2 = 0.25

NC = 2
NS = 16
L = 16

TILES_PER_BATCH = (NC * NS) // B
P_PER_TILE = -(-P // TILES_PER_BATCH)
NVEC = N // L
BUF = S + L
ROW = 3 * S


def _grouper_body(xyz_hbm, cen_hbm, out_hbm, xb, yb, zb, cenv, bufi, bufj,
                  ob, padb, sem):
    sid = lax.axis_index("s")
    cid = lax.axis_index("c")
    wid = sid * NC + cid
    b = wid // TILES_PER_BATCH
    g = wid % TILES_PER_BATCH

    pltpu.sync_copy(xyz_hbm.at[pl.ds(b * 3 * N, N)], xb)
    pltpu.sync_copy(xyz_hbm.at[pl.ds(b * 3 * N + N, N)], yb)
    pltpu.sync_copy(xyz_hbm.at[pl.ds(b * 3 * N + 2 * N, N)], zb)
    pltpu.sync_copy(cen_hbm, cenv)

    lanes = lax.iota(jnp.int32, 16)
    zeros16 = jnp.zeros((16,), jnp.int32)
    lane0 = lanes == 0

    def center_splats(p):
        psplat = jnp.broadcast_to(p, (16,))
        return (plsc.load_gather(cenv, [psplat]),
                plsc.load_gather(cenv, [32 + psplat]),
                plsc.load_gather(cenv, [64 + psplat]))

    def finish_pair(buf, cnt, cen, slot, p):
        cx, cy, cz = cen
        cnt_v = jnp.broadcast_to(cnt, (16,))
        v0 = buf[pl.ds(0, 16)]
        for jj in range(16):
            plsc.store_compressed(padb.at[pl.ds(jj, 16)], v0, mask=lane0)
        padidx = padb[pl.ds(0, 16)]

        base = slot * ROW
        for s2 in range(S // L):
            pos = s2 * L + lanes
            iv = buf[pl.ds(s2 * L, 16)]
            iv = jnp.where(pos < cnt_v, iv, padidx)
            ob[pl.ds(base + s2 * L, 16)] = plsc.load_gather(xb, [iv]) - cx
            ob[pl.ds(base + S + s2 * L, 16)] = plsc.load_gather(yb, [iv]) - cy
            ob[pl.ds(base + 2 * S + s2 * L, 16)] = (
                plsc.load_gather(zb, [iv]) - cz)

        pair_id = jnp.where(p < P, b * P + p, B * P)
        pltpu.async_copy(ob.at[pl.ds(base, ROW)],
                         out_hbm.at[pl.ds(pair_id * ROW, ROW)], sem)

    def pair_step(j, carry):
        pa = g * P_PER_TILE + 2 * j
        pb = pa + 1
        cena = center_splats(pa)
        cenb = center_splats(pb)
        cxa, cya, cza = cena
        cxb, cyb, czb = cenb

        bufi[pl.ds(0, 16)] = zeros16
        bufj[pl.ds(0, 16)] = zeros16

        def scan_step(i, carry2):
            ca, cb = carry2
            off = i * L
            vx = xb[pl.ds(off, 16)]
            vy = yb[pl.ds(off, 16)]
            vz = zb[pl.ds(off, 16)]
            idxv = off + lanes
            dxa = vx - cxa
            dya = vy - cya
            dza = vz - cza
            d2a = dxa * dxa + dya * dya + dza * dza
            ma = d2a < R2
            plsc.store_compressed(
                bufi.at[pl.ds(jnp.minimum(ca, S), 16)], idxv, mask=ma)
            dxb = vx - cxb
            dyb = vy - cyb
            dzb = vz - czb
            d2b = dxb * dxb + dyb * dyb + dzb * dzb
            mb = d2b < R2
            plsc.store_compressed(
                bufj.at[pl.ds(jnp.minimum(cb, S), 16)], idxv, mask=mb)
            return (ca + jnp.max(plsc.all_reduce_population_count(ma)),
                    cb + jnp.max(plsc.all_reduce_population_count(mb)))

        ca, cb = plsc.parallel_loop(
            0, NVEC, step=1, unroll=8,
            carry=(jnp.int32(0), jnp.int32(0)))(scan_step)

        finish_pair(bufi, ca, cena, 2 * j, pa)
        finish_pair(bufj, cb, cenb, 2 * j + 1, pb)
        return carry

    lax.fori_loop(0, (P_PER_TILE + 1) // 2, pair_step, jnp.int32(0))
    for _ in range(2 * ((P_PER_TILE + 1) // 2)):
        pltpu.make_async_copy(
            ob.at[pl.ds(0, ROW)],
            out_hbm.at[pl.ds(B * P * ROW, ROW)], sem).wait()


@jax.jit
def kernel(xyz, centers):
    xyz_t = jnp.transpose(xyz, (0, 2, 1)).reshape(-1)
    cen_pad = jnp.zeros((4, 32), jnp.float32)
    cen_t = lax.dynamic_update_slice(
        cen_pad, jnp.transpose(centers[0], (1, 0)), (0, 0)).reshape(-1)

    grouper = functools.partial(
        pl.kernel,
        out_type=jax.ShapeDtypeStruct(((B * P + 1) * ROW,), jnp.float32),
        mesh=plsc.VectorSubcoreMesh(
            core_axis_name="c", subcore_axis_name="s",
            num_cores=NC, num_subcores=NS),
        compiler_params=pltpu.CompilerParams(needs_layout_passes=False),
        scratch_types=[
            pltpu.VMEM((N,), jnp.float32),
            pltpu.VMEM((N,), jnp.float32),
            pltpu.VMEM((N,), jnp.float32),
            pltpu.VMEM((128,), jnp.float32),
            pltpu.VMEM((BUF,), jnp.int32),
            pltpu.VMEM((BUF,), jnp.int32),
            pltpu.VMEM((2 * ((P_PER_TILE + 1) // 2) * ROW,), jnp.float32),
            pltpu.VMEM((128,), jnp.int32),
            pltpu.SemaphoreType.DMA,
        ],
    )(_grouper_body)

    out = grouper(xyz_t, cen_t)
    out = out[: B * P * ROW].reshape(B * P, 3, S)
    return jnp.transpose(out, (0, 2, 1))


# speedup vs baseline: 1.1614x; 1.1614x over previous
import functools

import jax
import jax.numpy as jnp
from jax import lax
from jax.experimental import pallas as pl
from jax.experimental.pallas import tpu as pltpu
from jax.experimental.pallas import tpu_sc as plsc

B = 8
N = 16384
P = 27
S = 512
R2 = 0.25

NC = 2
NS = 16
L = 16

TILES_PER_BATCH = (NC * NS) // B
P_PER_TILE = -(-P // TILES_PER_BATCH)
NVEC = N // L
BUF = S + L


def _grouper_body(xyz_hbm, cen_hbm, out_hbm, xb, yb, zb, cenv, bufi,
                  obx, oby, obz, padb):
    sid = lax.axis_index("s")
    cid = lax.axis_index("c")
    wid = sid * NC + cid
    b = wid // TILES_PER_BATCH
    g = wid % TILES_PER_BATCH

    pltpu.sync_copy(xyz_hbm.at[pl.ds(b * 3 * N, N)], xb)
    pltpu.sync_copy(xyz_hbm.at[pl.ds(b * 3 * N + N, N)], yb)
    pltpu.sync_copy(xyz_hbm.at[pl.ds(b * 3 * N + 2 * N, N)], zb)
    pltpu.sync_copy(cen_hbm, cenv)

    lanes = lax.iota(jnp.int32, 16)
    zeros16 = jnp.zeros((16,), jnp.int32)

    def do_pair(p):
        psplat = jnp.broadcast_to(p, (16,))
        cx = plsc.load_gather(cenv, [psplat])
        cy = plsc.load_gather(cenv, [32 + psplat])
        cz = plsc.load_gather(cenv, [64 + psplat])

        bufi[pl.ds(0, 16)] = zeros16

        def scan_step(i, cnt):
            off = i * L
            vx = xb[pl.ds(off, 16)]
            vy = yb[pl.ds(off, 16)]
            vz = zb[pl.ds(off, 16)]
            dx = vx - cx
            dy = vy - cy
            dz = vz - cz
            d2 = dx * dx + dy * dy + dz * dz
            m = d2 < R2
            plsc.store_compressed(
                bufi.at[pl.ds(jnp.minimum(cnt, S), 16)], off + lanes, mask=m)
            return cnt + jnp.max(plsc.all_reduce_population_count(m))

        cnt = plsc.parallel_loop(
            0, NVEC, step=1, unroll=8, carry=jnp.int32(0))(scan_step)
        cnt_v = jnp.broadcast_to(cnt, (16,))

        v0 = bufi[pl.ds(0, 16)]
        lane0 = lanes == 0
        for j in range(16):
            plsc.store_compressed(padb.at[pl.ds(j, 16)], v0, mask=lane0)
        padidx = padb[pl.ds(0, 16)]
        for s2 in range(S // L):
            pos = s2 * L + lanes
            iv = bufi[pl.ds(s2 * L, 16)]
            iv = jnp.where(pos < cnt_v, iv, padidx)
            obx[pl.ds(s2 * L, 16)] = plsc.load_gather(xb, [iv]) - cx
            oby[pl.ds(s2 * L, 16)] = plsc.load_gather(yb, [iv]) - cy
            obz[pl.ds(s2 * L, 16)] = plsc.load_gather(zb, [iv]) - cz

        pair_id = b * P + p
        pltpu.sync_copy(obx, out_hbm.at[pl.ds(pair_id * S, S)])
        pltpu.sync_copy(oby, out_hbm.at[pl.ds((B * P + pair_id) * S, S)])
        pltpu.sync_copy(obz, out_hbm.at[pl.ds((2 * B * P + pair_id) * S, S)])

    def pair_step(j, carry):
        p = g * P_PER_TILE + j

        @pl.when(p < P)
        def _():
            do_pair(p)

        return carry

    lax.fori_loop(0, P_PER_TILE, pair_step, jnp.int32(0))


@jax.jit
def kernel(xyz, centers):
    xyz_t = jnp.transpose(xyz, (0, 2, 1)).reshape(-1)
    cen_pad = jnp.zeros((4, 32), jnp.float32)
    cen_t = lax.dynamic_update_slice(
        cen_pad, jnp.transpose(centers[0], (1, 0)), (0, 0)).reshape(-1)

    grouper = functools.partial(
        pl.kernel,
        out_type=jax.ShapeDtypeStruct((3 * B * P * S,), jnp.float32),
        mesh=plsc.VectorSubcoreMesh(
            core_axis_name="c", subcore_axis_name="s",
            num_cores=NC, num_subcores=NS),
        compiler_params=pltpu.CompilerParams(needs_layout_passes=False),
        scratch_types=[
            pltpu.VMEM((N,), jnp.float32),
            pltpu.VMEM((N,), jnp.float32),
            pltpu.VMEM((N,), jnp.float32),
            pltpu.VMEM((128,), jnp.float32),
            pltpu.VMEM((BUF,), jnp.int32),
            pltpu.VMEM((S,), jnp.float32),
            pltpu.VMEM((S,), jnp.float32),
            pltpu.VMEM((S,), jnp.float32),
            pltpu.VMEM((128,), jnp.int32),
        ],
    )(_grouper_body)

    out = grouper(xyz_t, cen_t)
    out = out.reshape(3, B * P, S)
    return jnp.transpose(out, (1, 2, 0))


# speedup vs baseline: 1.1831x; 1.0187x over previous
import functools

import jax
import jax.numpy as jnp
from jax import lax
from jax.experimental import pallas as pl
from jax.experimental.pallas import tpu as pltpu
from jax.experimental.pallas import tpu_sc as plsc

B = 8
N = 16384
P = 27
S = 512
R2 = 0.25

NC = 2
NS = 16
L = 16

TILES_PER_BATCH = (NC * NS) // B
P_PER_TILE = -(-P // TILES_PER_BATCH)
NVEC = N // L
BUF = N + L


def _grouper_body(xyz_hbm, cen_hbm, out_hbm, xb, yb, zb, cenv, bufi,
                  obx, oby, obz, padb):
    sid = lax.axis_index("s")
    cid = lax.axis_index("c")
    wid = sid * NC + cid
    b = wid // TILES_PER_BATCH
    g = wid % TILES_PER_BATCH

    pltpu.sync_copy(xyz_hbm.at[pl.ds(b * 3 * N, N)], xb)
    pltpu.sync_copy(xyz_hbm.at[pl.ds(b * 3 * N + N, N)], yb)
    pltpu.sync_copy(xyz_hbm.at[pl.ds(b * 3 * N + 2 * N, N)], zb)
    pltpu.sync_copy(cen_hbm, cenv)

    lanes = lax.iota(jnp.int32, 16)
    zeros16 = jnp.zeros((16,), jnp.int32)

    def do_pair(p):
        psplat = jnp.broadcast_to(p, (16,))
        cx = plsc.load_gather(cenv, [psplat])
        cy = plsc.load_gather(cenv, [32 + psplat])
        cz = plsc.load_gather(cenv, [64 + psplat])

        bufi[pl.ds(0, 16)] = zeros16

        def scan_step(i, cnt):
            off = i * L
            vx = xb[pl.ds(off, 16)]
            vy = yb[pl.ds(off, 16)]
            vz = zb[pl.ds(off, 16)]
            dx = vx - cx
            dy = vy - cy
            dz = vz - cz
            d2 = dx * dx + dy * dy + dz * dz
            m = d2 < R2
            plsc.store_compressed(
                bufi.at[pl.ds(cnt, 16)], off + lanes, mask=m)
            return cnt + jnp.max(plsc.all_reduce_population_count(m))

        cnt = plsc.parallel_loop(
            0, NVEC, step=1, unroll=8, carry=jnp.int32(0))(scan_step)
        cnt_v = jnp.broadcast_to(cnt, (16,))

        v0 = bufi[pl.ds(0, 16)]
        lane0 = lanes == 0
        for j in range(16):
            plsc.store_compressed(padb.at[pl.ds(j, 16)], v0, mask=lane0)
        padidx = padb[pl.ds(0, 16)]
        for s2 in range(S // L):
            pos = s2 * L + lanes
            iv = bufi[pl.ds(s2 * L, 16)]
            iv = jnp.where(pos < cnt_v, iv, padidx)
            obx[pl.ds(s2 * L, 16)] = plsc.load_gather(xb, [iv]) - cx
            oby[pl.ds(s2 * L, 16)] = plsc.load_gather(yb, [iv]) - cy
            obz[pl.ds(s2 * L, 16)] = plsc.load_gather(zb, [iv]) - cz

        pair_id = b * P + p
        pltpu.sync_copy(obx, out_hbm.at[pl.ds(pair_id * S, S)])
        pltpu.sync_copy(oby, out_hbm.at[pl.ds((B * P + pair_id) * S, S)])
        pltpu.sync_copy(obz, out_hbm.at[pl.ds((2 * B * P + pair_id) * S, S)])

    def pair_step(j, carry):
        p = g * P_PER_TILE + j

        @pl.when(p < P)
        def _():
            do_pair(p)

        return carry

    lax.fori_loop(0, P_PER_TILE, pair_step, jnp.int32(0))


@jax.jit
def kernel(xyz, centers):
    xyz_t = jnp.transpose(xyz, (0, 2, 1)).reshape(-1)
    cen_pad = jnp.zeros((4, 32), jnp.float32)
    cen_t = lax.dynamic_update_slice(
        cen_pad, jnp.transpose(centers[0], (1, 0)), (0, 0)).reshape(-1)

    grouper = functools.partial(
        pl.kernel,
        out_type=jax.ShapeDtypeStruct((3 * B * P * S,), jnp.float32),
        mesh=plsc.VectorSubcoreMesh(
            core_axis_name="c", subcore_axis_name="s",
            num_cores=NC, num_subcores=NS),
        compiler_params=pltpu.CompilerParams(needs_layout_passes=False),
        scratch_types=[
            pltpu.VMEM((N,), jnp.float32),
            pltpu.VMEM((N,), jnp.float32),
            pltpu.VMEM((N,), jnp.float32),
            pltpu.VMEM((128,), jnp.float32),
            pltpu.VMEM((BUF,), jnp.int32),
            pltpu.VMEM((S,), jnp.float32),
            pltpu.VMEM((S,), jnp.float32),
            pltpu.VMEM((S,), jnp.float32),
            pltpu.VMEM((128,), jnp.int32),
        ],
    )(_grouper_body)

    out = grouper(xyz_t, cen_t)
    out = out.reshape(3, B * P, S)
    return jnp.transpose(out, (1, 2, 0))
